# baseline (speedup 1.0000x reference)
import jax
import jax.numpy as jnp
from jax.experimental import pallas as pl
from jax.experimental.pallas import tpu as pltpu

S_BLK = 1024


def _add_kernel(x_ref, pe_ref, o_ref):
    o_ref[...] = x_ref[...] + pe_ref[...]


def kernel(x, pe_table):
    batch, seq_len, embed_dim = x.shape
    n_s = seq_len // S_BLK
    return pl.pallas_call(
        _add_kernel,
        grid=(n_s, batch),
        in_specs=[
            pl.BlockSpec((1, S_BLK, embed_dim), lambda s, b: (b, s, 0)),
            pl.BlockSpec((S_BLK, embed_dim), lambda s, b: (s, 0)),
        ],
        out_specs=pl.BlockSpec((1, S_BLK, embed_dim), lambda s, b: (b, s, 0)),
        out_shape=jax.ShapeDtypeStruct(x.shape, x.dtype),
        compiler_params=pltpu.CompilerParams(
            dimension_semantics=("arbitrary", "arbitrary"),
        ),
    )(x, pe_table)
